# baseline (speedup 1.0000x reference)
import functools

import jax
import jax.numpy as jnp
from jax.experimental import pallas as pl
from jax.experimental.pallas import tpu as pltpu

_BLK_H = 32
_N_STREAMS = 4


def _body(tc_ref, *refs, nh_total, b_total, n_total, blk_h, n_streams):
    i = pl.program_id(0)
    jb = pl.program_id(1)

    f_refs = refs[:n_streams]
    cen_ref, mem_ref, out_ref, ld_ref, acc_ref = refs[n_streams:]

    cen = cen_ref[...]
    c2 = jnp.sum(cen * cen, axis=1, keepdims=True)
    tc = tc_ref[0, 0]

    sub = blk_h // n_streams
    sum_a = 0.0
    cnt_a = 0.0
    sum_all = 0.0
    for half, fref in enumerate(f_refs):
        fblk = fref[0]
        f2 = jnp.sum(fblk * fblk, axis=0)
        ft = jnp.transpose(fblk, (1, 0, 2))

        rows = []
        for hh in range(sub):
            cross = jax.lax.dot_general(
                cen, ft[hh], (((1,), (0,)), ((), ())),
                preferred_element_type=jnp.float32)
            g = c2 - 2.0 * cross
            rows.append(jnp.min(g, axis=0, keepdims=True))
        gmin = jnp.concatenate(rows, axis=0)

        dmin = jnp.sqrt(jnp.maximum(f2 + gmin, 1e-12))
        mask = dmin > tc
        out_ref[0, :, half * sub:(half + 1) * sub, :] = jnp.where(
            mask[None], mem_ref[:, half * sub:(half + 1) * sub, :], fblk)

        sum_a = sum_a + jnp.sum(jnp.where(mask, dmin, 0.0))
        cnt_a = cnt_a + jnp.sum(mask.astype(jnp.float32))
        sum_all = sum_all + jnp.sum(dmin)

    @pl.when((i == 0) & (jb == 0))
    def _init():
        acc_ref[0] = 0.0
        acc_ref[1] = 0.0
        acc_ref[2] = 0.0

    acc_ref[0] += sum_a
    acc_ref[1] += cnt_a
    acc_ref[2] += sum_all

    @pl.when((i == nh_total - 1) & (jb == b_total - 1))
    def _fin():
        sa = acc_ref[0]
        ca = acc_ref[1]
        sn = acc_ref[2] - sa
        cn = jnp.float32(n_total) - ca
        mean_ano = sa / jnp.maximum(ca, 1.0)
        mean_nor = sn / jnp.maximum(cn, 1.0)
        ld_ref[0, 0] = jnp.where(ca > 0.0, mean_nor / (mean_ano + 0.001),
                                 mean_nor)


def kernel(f, center, Tc, memory, is_object):
    b, c, h, w = f.shape
    k = center.shape[0]
    mem_r = memory.reshape(c, h, w)
    tc_arr = jnp.asarray(Tc, dtype=jnp.float32).reshape(1, 1)
    nh = h // _BLK_H

    ns = _N_STREAMS
    sub = _BLK_H // ns
    body = functools.partial(_body, nh_total=nh, b_total=b, n_total=b * h * w,
                             blk_h=_BLK_H, n_streams=ns)
    f_specs = [
        pl.BlockSpec((1, c, sub, w),
                     functools.partial(lambda s, i, jb: (jb, 0, ns * i + s, 0), s))
        for s in range(ns)
    ]
    out, ld = pl.pallas_call(
        body,
        grid=(nh, b),
        in_specs=[
            pl.BlockSpec(memory_space=pltpu.SMEM),
            *f_specs,
            pl.BlockSpec((k, c), lambda i, jb: (0, 0)),
            pl.BlockSpec((c, _BLK_H, w), lambda i, jb: (0, i, 0)),
        ],
        out_specs=[
            pl.BlockSpec((1, c, _BLK_H, w), lambda i, jb: (jb, 0, i, 0)),
            pl.BlockSpec(memory_space=pltpu.SMEM),
        ],
        out_shape=[
            jax.ShapeDtypeStruct((b, c, h, w), jnp.float32),
            jax.ShapeDtypeStruct((1, 1), jnp.float32),
        ],
        scratch_shapes=[pltpu.SMEM((4,), jnp.float32)],
        compiler_params=pltpu.CompilerParams(
            dimension_semantics=("arbitrary", "arbitrary")),
    )(tc_arr, *([f] * ns), center, mem_r)

    return out, ld[0, 0]


# speedup vs baseline: 1.2300x; 1.2300x over previous
import functools

import jax
import jax.numpy as jnp
from jax.experimental import pallas as pl
from jax.experimental.pallas import tpu as pltpu

_BLK_H = 128
_N_STREAMS = 4


def _body(tc_ref, *refs, nh_total, b_total, n_total, blk_h, n_streams):
    i = pl.program_id(0)
    jb = pl.program_id(1)

    f_refs = refs[:n_streams]
    cen_ref, mem_ref, out_ref, ld_ref, acc_ref = refs[n_streams:]

    cen = cen_ref[...]
    c2 = jnp.sum(cen * cen, axis=1, keepdims=True)
    tc = tc_ref[0, 0]

    sub = blk_h // n_streams
    sum_a = 0.0
    cnt_a = 0.0
    sum_all = 0.0
    for half, fref in enumerate(f_refs):
        fblk = fref[0]
        f2 = jnp.sum(fblk * fblk, axis=0)
        ft = jnp.transpose(fblk, (1, 0, 2))

        rows = []
        for hh in range(sub):
            cross = jax.lax.dot_general(
                cen, ft[hh], (((1,), (0,)), ((), ())),
                preferred_element_type=jnp.float32)
            g = c2 - 2.0 * cross
            rows.append(jnp.min(g, axis=0, keepdims=True))
        gmin = jnp.concatenate(rows, axis=0)

        dmin = jnp.sqrt(jnp.maximum(f2 + gmin, 1e-12))
        mask = dmin > tc
        out_ref[0, :, half * sub:(half + 1) * sub, :] = jnp.where(
            mask[None], mem_ref[:, half * sub:(half + 1) * sub, :], fblk)

        sum_a = sum_a + jnp.sum(jnp.where(mask, dmin, 0.0))
        cnt_a = cnt_a + jnp.sum(mask.astype(jnp.float32))
        sum_all = sum_all + jnp.sum(dmin)

    @pl.when((i == 0) & (jb == 0))
    def _init():
        acc_ref[0] = 0.0
        acc_ref[1] = 0.0
        acc_ref[2] = 0.0

    acc_ref[0] += sum_a
    acc_ref[1] += cnt_a
    acc_ref[2] += sum_all

    @pl.when((i == nh_total - 1) & (jb == b_total - 1))
    def _fin():
        sa = acc_ref[0]
        ca = acc_ref[1]
        sn = acc_ref[2] - sa
        cn = jnp.float32(n_total) - ca
        mean_ano = sa / jnp.maximum(ca, 1.0)
        mean_nor = sn / jnp.maximum(cn, 1.0)
        ld_ref[0, 0] = jnp.where(ca > 0.0, mean_nor / (mean_ano + 0.001),
                                 mean_nor)


def kernel(f, center, Tc, memory, is_object):
    b, c, h, w = f.shape
    k = center.shape[0]
    mem_r = memory.reshape(c, h, w)
    tc_arr = jnp.asarray(Tc, dtype=jnp.float32).reshape(1, 1)
    nh = h // _BLK_H

    ns = _N_STREAMS
    sub = _BLK_H // ns
    body = functools.partial(_body, nh_total=nh, b_total=b, n_total=b * h * w,
                             blk_h=_BLK_H, n_streams=ns)
    f_specs = [
        pl.BlockSpec((1, c, sub, w),
                     functools.partial(lambda s, i, jb: (jb, 0, ns * i + s, 0), s))
        for s in range(ns)
    ]
    out, ld = pl.pallas_call(
        body,
        grid=(nh, b),
        in_specs=[
            pl.BlockSpec(memory_space=pltpu.SMEM),
            *f_specs,
            pl.BlockSpec((k, c), lambda i, jb: (0, 0)),
            pl.BlockSpec((c, _BLK_H, w), lambda i, jb: (0, i, 0)),
        ],
        out_specs=[
            pl.BlockSpec((1, c, _BLK_H, w), lambda i, jb: (jb, 0, i, 0)),
            pl.BlockSpec(memory_space=pltpu.SMEM),
        ],
        out_shape=[
            jax.ShapeDtypeStruct((b, c, h, w), jnp.float32),
            jax.ShapeDtypeStruct((1, 1), jnp.float32),
        ],
        scratch_shapes=[pltpu.SMEM((4,), jnp.float32)],
        compiler_params=pltpu.CompilerParams(
            dimension_semantics=("arbitrary", "arbitrary")),
    )(tc_arr, *([f] * ns), center, mem_r)

    return out, ld[0, 0]


# speedup vs baseline: 1.2558x; 1.0209x over previous
import functools

import jax
import jax.numpy as jnp
from jax.experimental import pallas as pl
from jax.experimental.pallas import tpu as pltpu

_BLK_H = 128
_N_STREAMS = 8


def _body(tc_ref, *refs, nh_total, b_total, n_total, blk_h, n_streams):
    i = pl.program_id(0)
    jb = pl.program_id(1)

    f_refs = refs[:n_streams]
    cen_ref, mem_ref, out_ref, ld_ref, acc_ref = refs[n_streams:]

    cen = cen_ref[...]
    c2 = jnp.sum(cen * cen, axis=1, keepdims=True)
    tc = tc_ref[0, 0]

    sub = blk_h // n_streams
    sum_a = 0.0
    cnt_a = 0.0
    sum_all = 0.0
    for half, fref in enumerate(f_refs):
        fblk = fref[0]
        f2 = jnp.sum(fblk * fblk, axis=0)
        ft = jnp.transpose(fblk, (1, 0, 2))

        rows = []
        for hh in range(sub):
            cross = jax.lax.dot_general(
                cen, ft[hh], (((1,), (0,)), ((), ())),
                preferred_element_type=jnp.float32)
            g = c2 - 2.0 * cross
            rows.append(jnp.min(g, axis=0, keepdims=True))
        gmin = jnp.concatenate(rows, axis=0)

        dmin = jnp.sqrt(jnp.maximum(f2 + gmin, 1e-12))
        mask = dmin > tc
        out_ref[0, :, half * sub:(half + 1) * sub, :] = jnp.where(
            mask[None], mem_ref[:, half * sub:(half + 1) * sub, :], fblk)

        sum_a = sum_a + jnp.sum(jnp.where(mask, dmin, 0.0))
        cnt_a = cnt_a + jnp.sum(mask.astype(jnp.float32))
        sum_all = sum_all + jnp.sum(dmin)

    @pl.when((i == 0) & (jb == 0))
    def _init():
        acc_ref[0] = 0.0
        acc_ref[1] = 0.0
        acc_ref[2] = 0.0

    acc_ref[0] += sum_a
    acc_ref[1] += cnt_a
    acc_ref[2] += sum_all

    @pl.when((i == nh_total - 1) & (jb == b_total - 1))
    def _fin():
        sa = acc_ref[0]
        ca = acc_ref[1]
        sn = acc_ref[2] - sa
        cn = jnp.float32(n_total) - ca
        mean_ano = sa / jnp.maximum(ca, 1.0)
        mean_nor = sn / jnp.maximum(cn, 1.0)
        ld_ref[0, 0] = jnp.where(ca > 0.0, mean_nor / (mean_ano + 0.001),
                                 mean_nor)


def kernel(f, center, Tc, memory, is_object):
    b, c, h, w = f.shape
    k = center.shape[0]
    mem_r = memory.reshape(c, h, w)
    tc_arr = jnp.asarray(Tc, dtype=jnp.float32).reshape(1, 1)
    nh = h // _BLK_H

    ns = _N_STREAMS
    sub = _BLK_H // ns
    body = functools.partial(_body, nh_total=nh, b_total=b, n_total=b * h * w,
                             blk_h=_BLK_H, n_streams=ns)
    f_specs = [
        pl.BlockSpec((1, c, sub, w),
                     functools.partial(lambda s, i, jb: (jb, 0, ns * i + s, 0), s))
        for s in range(ns)
    ]
    out, ld = pl.pallas_call(
        body,
        grid=(nh, b),
        in_specs=[
            pl.BlockSpec(memory_space=pltpu.SMEM),
            *f_specs,
            pl.BlockSpec((k, c), lambda i, jb: (0, 0)),
            pl.BlockSpec((c, _BLK_H, w), lambda i, jb: (0, i, 0)),
        ],
        out_specs=[
            pl.BlockSpec((1, c, _BLK_H, w), lambda i, jb: (jb, 0, i, 0)),
            pl.BlockSpec(memory_space=pltpu.SMEM),
        ],
        out_shape=[
            jax.ShapeDtypeStruct((b, c, h, w), jnp.float32),
            jax.ShapeDtypeStruct((1, 1), jnp.float32),
        ],
        scratch_shapes=[pltpu.SMEM((4,), jnp.float32)],
        compiler_params=pltpu.CompilerParams(
            dimension_semantics=("arbitrary", "arbitrary")),
    )(tc_arr, *([f] * ns), center, mem_r)

    return out, ld[0, 0]
